# baseline (speedup 1.0000x reference)
import jax
import jax.numpy as jnp
from jax.experimental import pallas as pl
from jax.experimental.pallas import tpu as pltpu

N_TOK = 8192
DIM = 4096
BM = 256
BN = 1024


def _mm_kernel(x_ref, w_ref, b_ref, o_ref, wb_ref):
    m = pl.program_id(1)

    @pl.when(m == 0)
    def _cast_panel():
        wb_ref[...] = w_ref[...].astype(jnp.bfloat16)

    xb = x_ref[...].astype(jnp.bfloat16)
    acc = jnp.dot(xb, wb_ref[...], preferred_element_type=jnp.float32)
    o_ref[...] = acc + b_ref[...]


def kernel(x, weight, bias):
    b2 = bias.reshape(1, DIM)
    return pl.pallas_call(
        _mm_kernel,
        grid=(DIM // BN, N_TOK // BM),
        in_specs=[
            pl.BlockSpec((BM, DIM), lambda n, m: (m, 0)),
            pl.BlockSpec((DIM, BN), lambda n, m: (0, n)),
            pl.BlockSpec((1, BN), lambda n, m: (0, n)),
        ],
        out_specs=pl.BlockSpec((BM, BN), lambda n, m: (m, n)),
        out_shape=jax.ShapeDtypeStruct((N_TOK, DIM), jnp.float32),
        scratch_shapes=[pltpu.VMEM((DIM, BN), jnp.bfloat16)],
        compiler_params=pltpu.CompilerParams(
            vmem_limit_bytes=63 * 1024 * 1024,
        ),
    )(x, weight, b2)


# speedup vs baseline: 1.0822x; 1.0822x over previous
import jax
import jax.numpy as jnp
from jax.experimental import pallas as pl
from jax.experimental.pallas import tpu as pltpu

N_TOK = 8192
DIM = 4096
BM = 512
BN = 2048


def _mm_kernel(x_ref, w_ref, b_ref, o_ref):
    xb = x_ref[...].astype(jnp.bfloat16)
    acc = jnp.dot(xb, w_ref[...], preferred_element_type=jnp.float32)
    o_ref[...] = acc + b_ref[...]


def kernel(x, weight, bias):
    wb = weight.astype(jnp.bfloat16)
    b2 = bias.reshape(1, DIM)
    return pl.pallas_call(
        _mm_kernel,
        grid=(DIM // BN, N_TOK // BM),
        in_specs=[
            pl.BlockSpec((BM, DIM), lambda n, m: (m, 0)),
            pl.BlockSpec((DIM, BN), lambda n, m: (0, n)),
            pl.BlockSpec((1, BN), lambda n, m: (0, n)),
        ],
        out_specs=pl.BlockSpec((BM, BN), lambda n, m: (m, n)),
        out_shape=jax.ShapeDtypeStruct((N_TOK, DIM), jnp.float32),
        compiler_params=pltpu.CompilerParams(
            allow_input_fusion=[False, True, False],
            vmem_limit_bytes=63 * 1024 * 1024,
        ),
    )(x, wb, b2)


# speedup vs baseline: 1.0827x; 1.0005x over previous
import jax
import jax.numpy as jnp
from jax.experimental import pallas as pl
from jax.experimental.pallas import tpu as pltpu

N_TOK = 8192
DIM = 4096
BM = 512
BN = 2048


def _mm_kernel(x_ref, w_ref, b_ref, o_ref):
    xb = x_ref[...].astype(jnp.bfloat16)
    acc = jnp.dot(xb, w_ref[...], preferred_element_type=jnp.float32)
    o_ref[...] = acc + b_ref[...]


def kernel(x, weight, bias):
    wb = weight.astype(jnp.bfloat16)
    b2 = bias.reshape(1, DIM)
    return pl.pallas_call(
        _mm_kernel,
        grid=(DIM // BN, N_TOK // BM),
        in_specs=[
            pl.BlockSpec((BM, DIM), lambda n, m: (m, 0)),
            pl.BlockSpec((DIM, BN), lambda n, m: (0, n)),
            pl.BlockSpec((1, BN), lambda n, m: (0, n)),
        ],
        out_specs=pl.BlockSpec((BM, BN), lambda n, m: (m, n)),
        out_shape=jax.ShapeDtypeStruct((N_TOK, DIM), jnp.float32),
        compiler_params=pltpu.CompilerParams(
            dimension_semantics=(pltpu.PARALLEL, pltpu.ARBITRARY),
            allow_input_fusion=[False, True, False],
            vmem_limit_bytes=63 * 1024 * 1024,
        ),
    )(x, wb, b2)
